# Initial kernel scaffold; baseline (speedup 1.0000x reference)
import jax
import jax.numpy as jnp
from jax.experimental import pallas as pl


def kernel(x):
    raise NotImplementedError("write your pallas kernel here")


# speedup vs baseline: 38.7076x; 38.7076x over previous
import jax
import jax.numpy as jnp
from jax.experimental import pallas as pl
from jax.experimental.pallas import tpu as pltpu

_ROWS = 32768
_COLS = 2048
_BLK_ROWS = 1024


def _absmax_kernel(x_ref, out_ref):
    i = pl.program_id(0)
    m = jnp.max(jnp.abs(x_ref[...]))

    @pl.when(i == 0)
    def _init():
        out_ref[0, 0] = m

    @pl.when(i > 0)
    def _acc():
        out_ref[0, 0] = jnp.maximum(out_ref[0, 0], m)


def _quant_kernel(xmax_ref, x_ref, out_ref):
    x_max = xmax_ref[0, 0]
    scale = x_max / 63.0
    x = x_ref[...]
    q = jnp.round(x / scale)
    q = jnp.clip(q, -64.0, 63.0)
    deq = q * scale
    out_ref[...] = jnp.where(x_max > 0.0, x + (deq - x), x)


def kernel(x):
    orig_shape = x.shape
    x2 = x.reshape(_ROWS, _COLS)
    grid = (_ROWS // _BLK_ROWS,)

    x_max = pl.pallas_call(
        _absmax_kernel,
        grid=grid,
        in_specs=[pl.BlockSpec((_BLK_ROWS, _COLS), lambda i: (i, 0))],
        out_specs=pl.BlockSpec(
            (1, 1), lambda i: (0, 0), memory_space=pltpu.SMEM
        ),
        out_shape=jax.ShapeDtypeStruct((1, 1), jnp.float32),
        compiler_params=pltpu.CompilerParams(
            dimension_semantics=("arbitrary",)
        ),
    )(x2)

    out = pl.pallas_call(
        _quant_kernel,
        grid=grid,
        in_specs=[
            pl.BlockSpec(memory_space=pltpu.SMEM),
            pl.BlockSpec((_BLK_ROWS, _COLS), lambda i: (i, 0)),
        ],
        out_specs=pl.BlockSpec((_BLK_ROWS, _COLS), lambda i: (i, 0)),
        out_shape=jax.ShapeDtypeStruct((_ROWS, _COLS), jnp.float32),
        compiler_params=pltpu.CompilerParams(
            dimension_semantics=("parallel",)
        ),
    )(x_max, x2)

    return out.reshape(orig_shape)
